# baseline (speedup 1.0000x reference)
import functools
import math

import jax
import jax.numpy as jnp
from jax import lax
from jax.experimental import pallas as pl
from jax.experimental.pallas import tpu as pltpu
from jax.experimental.pallas import tpu_sc as plsc

NC = 2
NS = 16
NW = NC * NS
LANES = 16

DMODEL = 64
CHUNK = 128
SCALE = math.sqrt(DMODEL)


def _make_emb(num_chunks_total):
    chunks_per_w = num_chunks_total // NW
    mesh = plsc.VectorSubcoreMesh(
        core_axis_name="c", subcore_axis_name="s", num_cores=NC, num_subcores=NS
    )

    @functools.partial(
        pl.kernel,
        out_type=jax.ShapeDtypeStruct((num_chunks_total, CHUNK, DMODEL), jnp.float32),
        mesh=mesh,
        scratch_types=[
            pltpu.VMEM((chunks_per_w, CHUNK), jnp.int32),
            pltpu.VMEM((CHUNK, DMODEL), jnp.float32),
            pltpu.SemaphoreType.DMA,
        ],
        compiler_params=pltpu.CompilerParams(use_tc_tiling_on_sc=False),
    )
    def emb(x_hbm, table_hbm, out_hbm, idx_v, rows_v, gsem):
        wid = lax.axis_index("s") * NC + lax.axis_index("c")
        base = wid * chunks_per_w
        pltpu.sync_copy(x_hbm.at[pl.ds(base, chunks_per_w)], idx_v)

        def step(j, carry):
            pltpu.async_copy(table_hbm.at[idx_v.at[j]], rows_v, gsem).wait()

            def srow(i, c2):
                for t in range(DMODEL // LANES):
                    sl = pl.ds(t * LANES, LANES)
                    rows_v[i, sl] = rows_v[i, sl] * SCALE
                return c2

            lax.fori_loop(0, CHUNK, srow, None)
            pltpu.sync_copy(rows_v, out_hbm.at[base + j])
            return carry

        lax.fori_loop(0, chunks_per_w, step, None)

    return emb


def kernel(x, table):
    b0, b1 = x.shape
    total = b0 * b1
    num_chunks = total // CHUNK
    xf = x.reshape(num_chunks, CHUNK).astype(jnp.int32)
    out = _make_emb(num_chunks)(xf, table)
    return out.reshape(b0, b1, DMODEL)


# speedup vs baseline: 1.0988x; 1.0988x over previous
import functools
import math

import jax
import jax.numpy as jnp
from jax import lax
from jax.experimental import pallas as pl
from jax.experimental.pallas import tpu as pltpu
from jax.experimental.pallas import tpu_sc as plsc

NC = 2
NS = 16
NW = NC * NS
LANES = 16

DMODEL = 64
CHUNK = 128
NBUF = 4
SCALE = math.sqrt(DMODEL)


def _make_emb(num_chunks_total):
    chunks_per_w = num_chunks_total // NW
    rounds = chunks_per_w // NBUF
    mesh = plsc.VectorSubcoreMesh(
        core_axis_name="c", subcore_axis_name="s", num_cores=NC, num_subcores=NS
    )

    @functools.partial(
        pl.kernel,
        out_type=jax.ShapeDtypeStruct((num_chunks_total, CHUNK, DMODEL), jnp.float32),
        mesh=mesh,
        scratch_types=[
            pltpu.VMEM((chunks_per_w, CHUNK), jnp.int32),
            pltpu.VMEM((NBUF, CHUNK, DMODEL), jnp.float32),
            pltpu.VMEM((NBUF, CHUNK, DMODEL), jnp.float32),
            pltpu.SemaphoreType.DMA((NBUF,)),
            pltpu.SemaphoreType.DMA((NBUF,)),
        ],
        compiler_params=pltpu.CompilerParams(use_tc_tiling_on_sc=False),
    )
    def emb(x_hbm, table_hbm, out_hbm, idx_v, gbuf, wbuf, gsem, wsem):
        wid = lax.axis_index("s") * NC + lax.axis_index("c")
        base = wid * chunks_per_w
        pltpu.sync_copy(x_hbm.at[pl.ds(base, chunks_per_w)], idx_v)

        def start_gather(c, b):
            pltpu.async_copy(table_hbm.at[idx_v.at[c]], gbuf.at[b], gsem.at[b])

        def wait_gather(b):
            pltpu.make_async_copy(
                table_hbm.at[pl.ds(0, CHUNK)], gbuf.at[b], gsem.at[b]
            ).wait()

        def start_write(c, b):
            pltpu.async_copy(wbuf.at[b], out_hbm.at[base + c], wsem.at[b])

        def wait_write(b):
            pltpu.make_async_copy(wbuf.at[b], out_hbm.at[0], wsem.at[b]).wait()

        def scale(b):
            def srow(i, c2):
                for t in range(DMODEL // LANES):
                    sl = pl.ds(t * LANES, LANES)
                    wbuf[b, i, sl] = gbuf[b, i, sl] * SCALE
                return c2

            lax.fori_loop(0, CHUNK, srow, None, unroll=4)

        for b in range(NBUF):
            start_gather(b, b)

        for b in range(NBUF):
            wait_gather(b)
            scale(b)
            start_gather(NBUF + b, b)
            start_write(b, b)

        def round_body(g, carry):
            for b in range(NBUF):
                c = g * NBUF + b
                wait_gather(b)
                wait_write(b)
                scale(b)
                start_gather(c + NBUF, b)
                start_write(c, b)
            return carry

        lax.fori_loop(1, rounds - 1, round_body, None)

        for b in range(NBUF):
            c = (rounds - 1) * NBUF + b
            wait_gather(b)
            wait_write(b)
            scale(b)
            start_write(c, b)

        for b in range(NBUF):
            wait_write(b)

    return emb


def kernel(x, table):
    b0, b1 = x.shape
    total = b0 * b1
    num_chunks = total // CHUNK
    xf = x.reshape(num_chunks, CHUNK).astype(jnp.int32)
    out = _make_emb(num_chunks)(xf, table)
    return out.reshape(b0, b1, DMODEL)
